# baseline (speedup 1.0000x reference)
import functools

import numpy as np

import jax
import jax.numpy as jnp
from jax import lax
from jax.experimental import pallas as pl
from jax.experimental.pallas import tpu as pltpu
from jax.experimental.pallas import tpu_sc as plsc

N_NODES = 10000
D_FEAT = 128
N_EDGES = 320000

NC = 2
NS = 16
NW = NC * NS
CH = 128
CPW = 80
HCH = 40
CPW0 = 80
CPW1 = 80
E_PAD = NW * CH * CPW
RPT = 632
AR = NS * RPT


def _sc_scatter(x, src2d, dst2d):
    mesh = plsc.VectorSubcoreMesh(core_axis_name="c", subcore_axis_name="s")

    @functools.partial(
        pl.kernel,
        out_type=[
            jax.ShapeDtypeStruct((NC * AR, D_FEAT), jnp.float32),
            jax.ShapeDtypeStruct((NC * AR,), jnp.float32),
        ],
        mesh=mesh,
        scratch_types=[
            pltpu.VMEM((HCH, CH), jnp.int32),
            pltpu.VMEM((HCH, CH), jnp.int32),
            pltpu.VMEM((CH, D_FEAT), jnp.float32),
            pltpu.VMEM((CH, D_FEAT), jnp.float32),
            pltpu.VMEM((CH,), jnp.float32),
            pltpu.VMEM((RPT,), jnp.float32),
            pltpu.VMEM_SHARED((AR, D_FEAT), jnp.float32),
            pltpu.VMEM_SHARED((AR,), jnp.float32),
            pltpu.SemaphoreType.DMA,
            pltpu.SemaphoreType.DMA,
            pltpu.SemaphoreType.DMA,
            pltpu.SemaphoreType.DMA,
            pltpu.SemaphoreType.DMA,
            pltpu.SemaphoreType.DMA,
        ],
    )
    def k(x_hbm, src_hbm, dst_hbm, psum_hbm, pdeg_hbm,
          src_v, dst_v, buf_a, buf_b, ones_v, zdeg_v, acc, dacc,
          sem_ga, sem_gb, sem_sa, sem_sb, sem_da, sem_db):
        c = lax.axis_index("c")
        s = lax.axis_index("s")
        w = s * NC + c

        zv = jnp.zeros((16,), jnp.float32)
        ov = jnp.ones((16,), jnp.float32)

        def fill_rows(r, carry):
            for t in range(D_FEAT // 16):
                buf_a[r, pl.ds(16 * t, 16)] = zv
            return carry

        lax.fori_loop(0, CH, fill_rows, 0)

        for t in range(CH // 16):
            ones_v[pl.ds(16 * t, 16)] = ov
        for t in range(RPT // 16):
            zdeg_v[pl.ds(16 * t, 16)] = zv
        zdeg_v[pl.ds(RPT - 16, 16)] = zv

        base = s * RPT
        for t in range(RPT // CH):
            pltpu.sync_copy(buf_a, acc.at[pl.ds(base + t * CH, CH)])
        rem = RPT - (RPT // CH) * CH
        pltpu.sync_copy(buf_a.at[pl.ds(0, rem)],
                        acc.at[pl.ds(base + RPT - rem, rem)])
        pltpu.sync_copy(zdeg_v, dacc.at[pl.ds(base, RPT)])

        base0 = jnp.where(c == 0, s * CPW0, NS * CPW0 + s * CPW1)
        pltpu.sync_copy(src_hbm.at[pl.ds(base0, HCH)], src_v)
        pltpu.sync_copy(dst_hbm.at[pl.ds(base0, HCH)], dst_v)

        plsc.subcore_barrier()

        def issue_g(j, buf, sem):
            return pltpu.async_copy(x_hbm.at[src_v.at[j]], buf, sem)

        def issue_s(j, buf, sem):
            return pltpu.async_copy(buf, acc.at[dst_v.at[j]], sem, add=True)

        def issue_d(j, sem):
            return pltpu.async_copy(ones_v, dacc.at[dst_v.at[j]], sem, add=True)

        def wait_g(buf, sem):
            pltpu.make_async_copy(x_hbm.at[src_v.at[0]], buf, sem).wait()

        def wait_s(buf, sem):
            pltpu.make_async_copy(buf, acc.at[dst_v.at[0]], sem).wait()

        def wait_d(sem):
            pltpu.make_async_copy(ones_v, dacc.at[dst_v.at[0]], sem).wait()

        def run_block(off, load_idx=True):
            if load_idx:
                pltpu.sync_copy(src_hbm.at[pl.ds(off, HCH)], src_v)
                pltpu.sync_copy(dst_hbm.at[pl.ds(off, HCH)], dst_v)

            issue_g(0, buf_a, sem_ga)
            wait_g(buf_a, sem_ga)
            issue_g(1, buf_b, sem_gb)
            issue_s(0, buf_a, sem_sa)
            issue_d(0, sem_da)
            wait_g(buf_b, sem_gb)
            wait_s(buf_a, sem_sa)
            wait_d(sem_da)
            issue_g(2, buf_a, sem_ga)
            issue_s(1, buf_b, sem_sb)
            issue_d(1, sem_db)

            def pair(i, carry):
                j0 = 2 * i
                j1 = 2 * i + 1
                wait_g(buf_a, sem_ga)
                wait_s(buf_b, sem_sb)
                wait_d(sem_db)
                issue_g(j1, buf_b, sem_gb)
                issue_s(j0, buf_a, sem_sa)
                issue_d(j0, sem_da)
                wait_g(buf_b, sem_gb)
                wait_s(buf_a, sem_sa)
                wait_d(sem_da)
                jn = jnp.where(j1 + 1 < HCH, j1 + 1, 0)
                issue_g(jn, buf_a, sem_ga)
                issue_s(j1, buf_b, sem_sb)
                issue_d(j1, sem_db)
                return carry

            lax.fori_loop(1, HCH // 2, pair, 0)

            wait_g(buf_a, sem_ga)
            wait_s(buf_b, sem_sb)
            wait_d(sem_db)

        for h in range(CPW0 // HCH):
            run_block(base0 + h * HCH, load_idx=h > 0)

        plsc.subcore_barrier()

        off = c * AR + base
        pltpu.sync_copy(acc.at[pl.ds(base, RPT)], psum_hbm.at[pl.ds(off, RPT)])
        pltpu.sync_copy(dacc.at[pl.ds(base, RPT)], zdeg_v)
        pltpu.sync_copy(zdeg_v, pdeg_hbm.at[pl.ds(off, RPT)])

    return k(x, src2d, dst2d)


def _tc_combine(psum, pdeg):
    def body(p_ref, d_ref, o_ref):
        s0 = p_ref[0:N_NODES, :]
        s1 = p_ref[AR:AR + N_NODES, :]
        deg = jnp.maximum(d_ref[0:N_NODES] + d_ref[AR:AR + N_NODES], 1.0)
        o_ref[...] = (s0 + s1) / deg.reshape(N_NODES, 1)

    return pl.pallas_call(
        body,
        out_shape=jax.ShapeDtypeStruct((N_NODES, D_FEAT), jnp.float32),
    )(psum, pdeg)


def kernel(x, edge_index):
    src = edge_index[0].astype(jnp.int32)
    dst = edge_index[1].astype(jnp.int32)
    pad = E_PAD - N_EDGES
    pad_iota = np.arange(pad, dtype=np.int32)
    src = jnp.concatenate([src, jnp.asarray(pad_iota % N_NODES)])
    dst = jnp.concatenate([dst, jnp.asarray(N_NODES + pad_iota % (AR - N_NODES))])
    src2d = src.reshape(NW * CPW, CH)
    dst2d = dst.reshape(NW * CPW, CH)
    psum, pdeg = _sc_scatter(x, src2d, dst2d)
    return _tc_combine(psum, pdeg)


# speedup vs baseline: 1.0075x; 1.0075x over previous
import functools

import numpy as np

import jax
import jax.numpy as jnp
from jax import lax
from jax.experimental import pallas as pl
from jax.experimental.pallas import tpu as pltpu
from jax.experimental.pallas import tpu_sc as plsc

N_NODES = 10000
D_FEAT = 128
N_EDGES = 320000

NC = 2
NS = 16
NW = NC * NS
CH = 128
CPW = 80
HCH = 40
CPW0 = 80
CPW1 = 80
E_PAD = NW * CH * CPW
RPT = 632
AR = NS * RPT


def _sc_scatter(x, src2d, dst2d):
    mesh = plsc.VectorSubcoreMesh(core_axis_name="c", subcore_axis_name="s")

    @functools.partial(
        pl.kernel,
        out_type=[
            jax.ShapeDtypeStruct((NC * AR, D_FEAT), jnp.float32),
            jax.ShapeDtypeStruct((NC * AR,), jnp.float32),
        ],
        mesh=mesh,
        scratch_types=[
            pltpu.VMEM((HCH, CH), jnp.int32),
            pltpu.VMEM((HCH, CH), jnp.int32),
            pltpu.VMEM((CH, D_FEAT), jnp.float32),
            pltpu.VMEM((CH, D_FEAT), jnp.float32),
            pltpu.VMEM((CH,), jnp.float32),
            pltpu.VMEM((RPT,), jnp.float32),
            pltpu.VMEM_SHARED((AR, D_FEAT), jnp.float32),
            pltpu.VMEM_SHARED((AR,), jnp.float32),
            pltpu.SemaphoreType.DMA,
            pltpu.SemaphoreType.DMA,
            pltpu.SemaphoreType.DMA,
            pltpu.SemaphoreType.DMA,
            pltpu.SemaphoreType.DMA,
            pltpu.SemaphoreType.DMA,
        ],
    )
    def k(x_hbm, src_hbm, dst_hbm, psum_hbm, pdeg_hbm,
          src_v, dst_v, buf_a, buf_b, ones_v, zdeg_v, acc, dacc,
          sem_ga, sem_gb, sem_sa, sem_sb, sem_da, sem_db):
        c = lax.axis_index("c")
        s = lax.axis_index("s")

        zv = jnp.zeros((16,), jnp.float32)
        ov = jnp.ones((16,), jnp.float32)

        def fill_rows(r, carry):
            for t in range(D_FEAT // 16):
                buf_a[r, pl.ds(16 * t, 16)] = zv
            return carry

        lax.fori_loop(0, CH, fill_rows, 0)

        for t in range(CH // 16):
            ones_v[pl.ds(16 * t, 16)] = ov
        for t in range(RPT // 16):
            zdeg_v[pl.ds(16 * t, 16)] = zv
        zdeg_v[pl.ds(RPT - 16, 16)] = zv

        base = s * RPT
        for t in range(RPT // CH):
            pltpu.sync_copy(buf_a, acc.at[pl.ds(base + t * CH, CH)])
        rem = RPT - (RPT // CH) * CH
        pltpu.sync_copy(buf_a.at[pl.ds(0, rem)],
                        acc.at[pl.ds(base + RPT - rem, rem)])
        pltpu.sync_copy(zdeg_v, dacc.at[pl.ds(base, RPT)])

        plsc.subcore_barrier()

        def issue_g(j, buf, sem):
            return pltpu.async_copy(x_hbm.at[src_v.at[j]], buf, sem)

        def issue_s(j, buf, sem):
            return pltpu.async_copy(buf, acc.at[dst_v.at[j]], sem, add=True)

        def issue_d(j, sem):
            return pltpu.async_copy(ones_v, dacc.at[dst_v.at[j]], sem, add=True)

        def wait_g(buf, sem):
            pltpu.make_async_copy(x_hbm.at[src_v.at[0]], buf, sem).wait()

        def wait_s(buf, sem):
            pltpu.make_async_copy(buf, acc.at[dst_v.at[0]], sem).wait()

        def wait_d(sem):
            pltpu.make_async_copy(ones_v, dacc.at[dst_v.at[0]], sem).wait()

        def run_block(off):
            pltpu.sync_copy(src_hbm.at[pl.ds(off, HCH)], src_v)
            pltpu.sync_copy(dst_hbm.at[pl.ds(off, HCH)], dst_v)

            issue_g(0, buf_a, sem_ga)
            wait_g(buf_a, sem_ga)
            issue_g(1, buf_b, sem_gb)
            issue_s(0, buf_a, sem_sa)
            issue_d(0, sem_da)
            wait_g(buf_b, sem_gb)
            wait_s(buf_a, sem_sa)
            wait_d(sem_da)
            issue_g(2, buf_a, sem_ga)
            issue_s(1, buf_b, sem_sb)
            issue_d(1, sem_db)

            def pair(i, carry):
                j0 = 2 * i
                j1 = 2 * i + 1
                wait_g(buf_a, sem_ga)
                wait_s(buf_b, sem_sb)
                wait_d(sem_db)
                issue_g(j1, buf_b, sem_gb)
                issue_s(j0, buf_a, sem_sa)
                issue_d(j0, sem_da)
                wait_g(buf_b, sem_gb)
                wait_s(buf_a, sem_sa)
                wait_d(sem_da)
                jn = jnp.where(j1 + 1 < HCH, j1 + 1, 0)
                issue_g(jn, buf_a, sem_ga)
                issue_s(j1, buf_b, sem_sb)
                issue_d(j1, sem_db)
                return carry

            lax.fori_loop(1, HCH // 2, pair, 0)

            wait_g(buf_a, sem_ga)
            wait_s(buf_b, sem_sb)
            wait_d(sem_db)

        @pl.when(c == 0)
        def _core0():
            for h in range(CPW0 // HCH):
                run_block(s * CPW0 + h * HCH)

        @pl.when(c == 1)
        def _core1():
            for h in range(CPW1 // HCH):
                run_block(NS * CPW0 + s * CPW1 + h * HCH)

        plsc.subcore_barrier()

        off = c * AR + base
        pltpu.sync_copy(acc.at[pl.ds(base, RPT)], psum_hbm.at[pl.ds(off, RPT)])
        pltpu.sync_copy(dacc.at[pl.ds(base, RPT)], zdeg_v)
        pltpu.sync_copy(zdeg_v, pdeg_hbm.at[pl.ds(off, RPT)])

    return k(x, src2d, dst2d)


def _tc_combine(psum, pdeg):
    def body(p_ref, d_ref, o_ref):
        s0 = p_ref[0:N_NODES, :]
        s1 = p_ref[AR:AR + N_NODES, :]
        deg = jnp.maximum(d_ref[0:N_NODES] + d_ref[AR:AR + N_NODES], 1.0)
        o_ref[...] = (s0 + s1) / deg.reshape(N_NODES, 1)

    return pl.pallas_call(
        body,
        out_shape=jax.ShapeDtypeStruct((N_NODES, D_FEAT), jnp.float32),
    )(psum, pdeg)


def kernel(x, edge_index):
    src = edge_index[0].astype(jnp.int32)
    dst = edge_index[1].astype(jnp.int32)
    pad = E_PAD - N_EDGES
    pad_iota = np.arange(pad, dtype=np.int32)
    src = jnp.concatenate([src, jnp.asarray(pad_iota % N_NODES)])
    dst = jnp.concatenate([dst, jnp.asarray(N_NODES + pad_iota % (AR - N_NODES))])
    src2d = src.reshape(NW * CPW, CH)
    dst2d = dst.reshape(NW * CPW, CH)
    psum, pdeg = _sc_scatter(x, src2d, dst2d)
    return _tc_combine(psum, pdeg)
